# baseline (speedup 1.0000x reference)
import functools

import jax
import jax.numpy as jnp
from jax import lax
from jax.experimental import pallas as pl
from jax.experimental.pallas import tpu as pltpu
from jax.experimental.pallas import tpu_sc as plsc

N = 10000
E = 320000
H = 128
HP = H // 2
L = 3
DE = 13
DEP = 16
G = 64

NC = 2
NS = 16
NW = NC * NS
EPT = E // NW
K = 128
NPAIR = 39
EPT_MAIN = 2 * K * NPAIR
KT = 16
TAIL0 = NW * EPT_MAIN
NP = 10240
ROWS_PT = NP // NS

_HI = jax.lax.Precision.HIGHEST


def _lin0_body(x_ref, wp_ref, bp_ref, h_ref):
    h_ref[...] = jnp.dot(x_ref[...], wp_ref[...]) + bp_ref[...]


def _msg_body(hd_ref, hs_ref, ea_ref, wd_ref, ws_ref, we_ref, bc_ref, m_ref):
    hd16 = hd_ref[...].astype(jnp.bfloat16)
    hs16 = hs_ref[...].astype(jnp.bfloat16)
    u = (jnp.dot(hd16, wd_ref[...], preferred_element_type=jnp.float32)
         + jnp.dot(hs16, ws_ref[...], preferred_element_type=jnp.float32)
         + jnp.dot(ea_ref[...], we_ref[...]) + bc_ref[...])
    uf = u[:, :H]
    us = u[:, H:]
    sg = 1.0 / (1.0 + jnp.exp(-uf))
    sp = jnp.maximum(us, 0.0) + jnp.log1p(jnp.exp(-jnp.abs(us)))
    m_ref[...] = sg * sp


def _bn(v, g, b):
    mu = jnp.mean(v, axis=0, keepdims=True)
    var = jnp.mean((v - mu) * (v - mu), axis=0, keepdims=True)
    return g * (v - mu) / jnp.sqrt(var + 1e-5) + b


def _update_body(h_ref, p_ref, gi_ref, bi_ref, go_ref, bo_ref, h2_ref):
    agg = _bn(p_ref[0, :N] + p_ref[1, :N], gi_ref[...], bi_ref[...])
    h2_ref[...] = jnp.maximum(_bn(h_ref[...] + agg, go_ref[...], bo_ref[...]), 0.0)


def _final_body(h_ref, p_ref, gi_ref, bi_ref, go_ref, bo_ref, batch_ref,
                w1_ref, b1_ref, w2_ref, b2_ref, o_ref):
    agg = _bn(p_ref[0, :N] + p_ref[1, :N], gi_ref[...], bi_ref[...])
    h = jnp.maximum(_bn(h_ref[...] + agg, go_ref[...], bo_ref[...]), 0.0)
    seg = lax.broadcasted_iota(jnp.int32, (G, N), 0)
    oht = (seg == batch_ref[...]).astype(jnp.float32)
    sums = jnp.dot(oht, h, precision=_HI)
    cnt = jnp.sum(oht, axis=1, keepdims=True)
    pooled = sums / jnp.maximum(cnt, 1.0)
    o1 = jnp.maximum(jnp.dot(pooled, w1_ref[...]) + b1_ref[...], 0.0)
    o_ref[...] = jnp.dot(o1, w2_ref[...]) + b2_ref[...]


def _gather_body(dst_ref, src_ref, hp_ref, td_ref, ts_ref,
                 idxd0, idxs0, idxd1, idxs1, idxdt, idxst,
                 bd0, bs0, bd1, bs1, sem0, sem1):
    c = lax.axis_index("c")
    s = lax.axis_index("s")
    wid = s * NC + c
    base = wid * EPT_MAIN

    def start(off, n, idxd, idxs, bd, bs, sem):
        pltpu.sync_copy(dst_ref.at[pl.ds(off, n)], idxd)
        pltpu.sync_copy(src_ref.at[pl.ds(off, n)], idxs)
        pltpu.async_copy(hp_ref.at[idxd], bd, sem)
        pltpu.async_copy(hp_ref.at[idxs], bs, sem)

    def finish(off, n, idxd, idxs, bd, bs, sem):
        pltpu.make_async_copy(hp_ref.at[idxd], bd, sem).wait()
        pltpu.make_async_copy(hp_ref.at[idxs], bs, sem).wait()
        pltpu.sync_copy(bd, td_ref.at[pl.ds(off, n)])
        pltpu.sync_copy(bs, ts_ref.at[pl.ds(off, n)])

    def body(j, carry):
        o0 = pl.multiple_of(base + (2 * j) * K, 8)
        o1 = pl.multiple_of(base + (2 * j + 1) * K, 8)
        start(o0, K, idxd0, idxs0, bd0, bs0, sem0)
        start(o1, K, idxd1, idxs1, bd1, bs1, sem1)
        finish(o0, K, idxd0, idxs0, bd0, bs0, sem0)
        finish(o1, K, idxd1, idxs1, bd1, bs1, sem1)
        return carry

    lax.fori_loop(0, NPAIR, body, 0)
    ot = pl.multiple_of(TAIL0 + wid * KT, 8)
    bdt = bd0.at[pl.ds(0, KT)]
    bst = bs0.at[pl.ds(0, KT)]
    start(ot, KT, idxdt, idxst, bdt, bst, sem0)
    finish(ot, KT, idxdt, idxst, bdt, bst, sem0)


def _scatter_body(dst_ref, m_ref, z_ref, out_ref, idx0, idx1, idxt, mb0, mb1,
                  sem0, sem1, agg_sh):
    c = lax.axis_index("c")
    s = lax.axis_index("s")
    wid = s * NC + c
    pltpu.sync_copy(z_ref.at[pl.ds(s * ROWS_PT, ROWS_PT)],
                    agg_sh.at[pl.ds(s * ROWS_PT, ROWS_PT)])
    plsc.subcore_barrier()
    base = wid * EPT_MAIN

    def start(off, n, idx, mb, sem):
        pltpu.sync_copy(dst_ref.at[pl.ds(off, n)], idx)
        pltpu.async_copy(m_ref.at[pl.ds(off, n)], mb, sem)

    def finish(off, n, idx, mb, sem):
        pltpu.make_async_copy(m_ref.at[pl.ds(off, n)], mb, sem).wait()
        pltpu.sync_copy(mb, agg_sh.at[idx], add=True)

    def body(j, carry):
        o0 = pl.multiple_of(base + (2 * j) * K, 8)
        o1 = pl.multiple_of(base + (2 * j + 1) * K, 8)
        start(o0, K, idx0, mb0, sem0)
        start(o1, K, idx1, mb1, sem1)
        finish(o0, K, idx0, mb0, sem0)
        finish(o1, K, idx1, mb1, sem1)
        return carry

    lax.fori_loop(0, NPAIR, body, 0)
    ot = pl.multiple_of(TAIL0 + wid * KT, 8)
    mbt = mb0.at[pl.ds(0, KT)]
    start(ot, KT, idxt, mbt, sem0)
    finish(ot, KT, idxt, mbt, sem0)
    plsc.subcore_barrier()
    pltpu.sync_copy(agg_sh.at[pl.ds(s * ROWS_PT, ROWS_PT)],
                    out_ref.at[c, pl.ds(s * ROWS_PT, ROWS_PT)])


@functools.lru_cache(maxsize=None)
def _sc_calls():
    mesh = plsc.VectorSubcoreMesh(core_axis_name="c", subcore_axis_name="s",
                                  num_cores=NC, num_subcores=NS)
    gather = pl.kernel(
        _gather_body,
        out_type=(jax.ShapeDtypeStruct((E, H), jnp.float32),
                  jax.ShapeDtypeStruct((E, H), jnp.float32)),
        mesh=mesh,
        scratch_types=[
            pltpu.VMEM((K,), jnp.int32),
            pltpu.VMEM((K,), jnp.int32),
            pltpu.VMEM((K,), jnp.int32),
            pltpu.VMEM((K,), jnp.int32),
            pltpu.VMEM((KT,), jnp.int32),
            pltpu.VMEM((KT,), jnp.int32),
            pltpu.VMEM((K, H), jnp.float32),
            pltpu.VMEM((K, H), jnp.float32),
            pltpu.VMEM((K, H), jnp.float32),
            pltpu.VMEM((K, H), jnp.float32),
            pltpu.SemaphoreType.DMA,
            pltpu.SemaphoreType.DMA,
        ],
    )
    scatter = pl.kernel(
        _scatter_body,
        out_type=jax.ShapeDtypeStruct((NC, NP, H), jnp.float32),
        mesh=mesh,
        scratch_types=[
            pltpu.VMEM((K,), jnp.int32),
            pltpu.VMEM((K,), jnp.int32),
            pltpu.VMEM((KT,), jnp.int32),
            pltpu.VMEM((K, H), jnp.float32),
            pltpu.VMEM((K, H), jnp.float32),
            pltpu.SemaphoreType.DMA,
            pltpu.SemaphoreType.DMA,
            pltpu.VMEM_SHARED((NP, H), jnp.float32),
        ],
    )
    return gather, scatter


_lin0_call = pl.pallas_call(
    _lin0_body,
    out_shape=jax.ShapeDtypeStruct((N, H), jnp.float32),
)

_BE = 1600

_msg_call = pl.pallas_call(
    _msg_body,
    grid=(E // _BE,),
    in_specs=[
        pl.BlockSpec((_BE, H), lambda i: (i, 0)),
        pl.BlockSpec((_BE, H), lambda i: (i, 0)),
        pl.BlockSpec((_BE, DEP), lambda i: (i, 0)),
        pl.BlockSpec((H, 2 * H), lambda i: (0, 0)),
        pl.BlockSpec((H, 2 * H), lambda i: (0, 0)),
        pl.BlockSpec((DEP, 2 * H), lambda i: (0, 0)),
        pl.BlockSpec((1, 2 * H), lambda i: (0, 0)),
    ],
    out_specs=pl.BlockSpec((_BE, H), lambda i: (i, 0)),
    out_shape=jax.ShapeDtypeStruct((E, H), jnp.float32),
)

_update_call = pl.pallas_call(
    _update_body,
    out_shape=jax.ShapeDtypeStruct((N, H), jnp.float32),
)

_final_call = pl.pallas_call(
    _final_body,
    out_shape=jax.ShapeDtypeStruct((G, 1), jnp.float32),
)


def kernel(x, edge_index, edge_attr, batch, Wp, bp, Wf, bf, Ws, bs,
           g_in, b_in, g_out, b_out, W1, b1, W2, b2):
    src = edge_index[0].astype(jnp.int32)
    dst = edge_index[1].astype(jnp.int32)
    ea = jnp.pad(edge_attr, ((0, 0), (0, DEP - DE)))
    zeros = jnp.zeros((NP, H), jnp.float32)
    batch_row = batch.astype(jnp.int32).reshape(1, N)

    wd = [jnp.concatenate([Wf[l][:H], Ws[l][:H]], axis=1).astype(jnp.bfloat16)
          for l in range(L)]
    wsr = [jnp.concatenate([Wf[l][H:2 * H], Ws[l][H:2 * H]],
                           axis=1).astype(jnp.bfloat16) for l in range(L)]
    we = [jnp.pad(jnp.concatenate([Wf[l][2 * H:], Ws[l][2 * H:]], axis=1),
                  ((0, DEP - DE), (0, 0))) for l in range(L)]
    bc = [jnp.concatenate([bf[l], bs[l]]).reshape(1, 2 * H) for l in range(L)]

    h = _lin0_call(x, Wp, bp.reshape(1, H))
    _gather_call, _scatter_call = _sc_calls()

    o = None
    for l in range(L):
        td, ts = _gather_call(dst, src, h)
        m = _msg_call(td, ts, ea, wd[l], wsr[l], we[l], bc[l])
        parts = _scatter_call(dst, m, zeros)
        gi = g_in[l].reshape(1, H)
        bi = b_in[l].reshape(1, H)
        go = g_out[l].reshape(1, H)
        bo = b_out[l].reshape(1, H)
        if l < L - 1:
            h = _update_call(h, parts, gi, bi, go, bo)
        else:
            o = _final_call(h, parts, gi, bi, go, bo, batch_row,
                            W1, b1.reshape(1, H // 2), W2, b2.reshape(1, 1))
    return o


# speedup vs baseline: 1.0327x; 1.0327x over previous
import functools

import jax
import jax.numpy as jnp
from jax import lax
from jax.experimental import pallas as pl
from jax.experimental.pallas import tpu as pltpu
from jax.experimental.pallas import tpu_sc as plsc

N = 10000
E = 320000
H = 128
HP = H // 2
L = 3
DE = 13
DEP = 16
G = 64

NC = 2
NS = 16
NW = NC * NS
EPT = E // NW
EH = E // 2
K = 64
NPAIR = 39
EPT_MAIN = 2 * K * NPAIR
KT = 8
TAIL0 = NW * EPT_MAIN
NP = 10240
ROWS_PT = NP // NS

_HI = jax.lax.Precision.HIGHEST


def _lin0_body(x_ref, wp_ref, bp_ref, h_ref):
    h_ref[...] = jnp.dot(x_ref[...], wp_ref[...]) + bp_ref[...]


def _msg_body(hd_ref, hs_ref, ea_ref, wd_ref, ws_ref, we_ref, bc_ref, m_ref):
    hd16 = hd_ref[...].astype(jnp.bfloat16)
    hs16 = hs_ref[...].astype(jnp.bfloat16)
    u = (jnp.dot(hd16, wd_ref[...], preferred_element_type=jnp.float32)
         + jnp.dot(hs16, ws_ref[...], preferred_element_type=jnp.float32)
         + jnp.dot(ea_ref[...], we_ref[...]) + bc_ref[...])
    uf = u[:, :H]
    us = u[:, H:]
    sg = 1.0 / (1.0 + jnp.exp(-uf))
    sp = jnp.maximum(us, 0.0) + jnp.log1p(jnp.exp(-jnp.abs(us)))
    m_ref[...] = sg * sp


def _bn(v, g, b):
    mu = jnp.mean(v, axis=0, keepdims=True)
    var = jnp.mean((v - mu) * (v - mu), axis=0, keepdims=True)
    return g * (v - mu) / jnp.sqrt(var + 1e-5) + b


def _update_body(h_ref, pa_ref, pb_ref, gi_ref, bi_ref, go_ref, bo_ref, h2_ref):
    agg = _bn((pa_ref[0, :N] + pa_ref[1, :N]) + (pb_ref[0, :N] + pb_ref[1, :N]),
              gi_ref[...], bi_ref[...])
    h2_ref[...] = jnp.maximum(_bn(h_ref[...] + agg, go_ref[...], bo_ref[...]), 0.0)


def _final_body(h_ref, pa_ref, pb_ref, gi_ref, bi_ref, go_ref, bo_ref, batch_ref,
                w1_ref, b1_ref, w2_ref, b2_ref, o_ref):
    agg = _bn((pa_ref[0, :N] + pa_ref[1, :N]) + (pb_ref[0, :N] + pb_ref[1, :N]),
              gi_ref[...], bi_ref[...])
    h = jnp.maximum(_bn(h_ref[...] + agg, go_ref[...], bo_ref[...]), 0.0)
    seg = lax.broadcasted_iota(jnp.int32, (G, N), 0)
    oht = (seg == batch_ref[...]).astype(jnp.float32)
    sums = jnp.dot(oht, h, precision=_HI)
    cnt = jnp.sum(oht, axis=1, keepdims=True)
    pooled = sums / jnp.maximum(cnt, 1.0)
    o1 = jnp.maximum(jnp.dot(pooled, w1_ref[...]) + b1_ref[...], 0.0)
    o_ref[...] = jnp.dot(o1, w2_ref[...]) + b2_ref[...]


def _gather_body(dst_ref, src_ref, hp_ref, td_ref, ts_ref,
                 idxd0, idxs0, idxd1, idxs1, idxdt, idxst,
                 bd0, bs0, bd1, bs1, sem0, sem1):
    c = lax.axis_index("c")
    s = lax.axis_index("s")
    wid = s * NC + c
    base = wid * EPT_MAIN

    def start(off, n, idxd, idxs, bd, bs, sem):
        pltpu.sync_copy(dst_ref.at[pl.ds(off, n)], idxd)
        pltpu.sync_copy(src_ref.at[pl.ds(off, n)], idxs)
        pltpu.async_copy(hp_ref.at[idxd], bd, sem)
        pltpu.async_copy(hp_ref.at[idxs], bs, sem)

    def finish(off, n, idxd, idxs, bd, bs, sem):
        pltpu.make_async_copy(hp_ref.at[idxd], bd, sem).wait()
        pltpu.make_async_copy(hp_ref.at[idxs], bs, sem).wait()
        pltpu.sync_copy(bd, td_ref.at[pl.ds(off, n)])
        pltpu.sync_copy(bs, ts_ref.at[pl.ds(off, n)])

    def body(j, carry):
        o0 = pl.multiple_of(base + (2 * j) * K, 8)
        o1 = pl.multiple_of(base + (2 * j + 1) * K, 8)
        start(o0, K, idxd0, idxs0, bd0, bs0, sem0)
        start(o1, K, idxd1, idxs1, bd1, bs1, sem1)
        finish(o0, K, idxd0, idxs0, bd0, bs0, sem0)
        finish(o1, K, idxd1, idxs1, bd1, bs1, sem1)
        return carry

    lax.fori_loop(0, NPAIR, body, 0)
    ot = pl.multiple_of(TAIL0 + wid * KT, 8)
    bdt = bd0.at[pl.ds(0, KT)]
    bst = bs0.at[pl.ds(0, KT)]
    start(ot, KT, idxdt, idxst, bdt, bst, sem0)
    finish(ot, KT, idxdt, idxst, bdt, bst, sem0)


def _scatter_body(dst_ref, m_ref, z_ref, out_ref, idx0, idx1, idxt, mb0, mb1,
                  sem0, sem1, agg_sh):
    c = lax.axis_index("c")
    s = lax.axis_index("s")
    wid = s * NC + c
    pltpu.sync_copy(z_ref.at[pl.ds(s * ROWS_PT, ROWS_PT)],
                    agg_sh.at[pl.ds(s * ROWS_PT, ROWS_PT)])
    plsc.subcore_barrier()
    base = wid * EPT_MAIN

    def start(off, n, idx, mb, sem):
        pltpu.sync_copy(dst_ref.at[pl.ds(off, n)], idx)
        pltpu.async_copy(m_ref.at[pl.ds(off, n)], mb, sem)

    def finish(off, n, idx, mb, sem):
        pltpu.make_async_copy(m_ref.at[pl.ds(off, n)], mb, sem).wait()
        pltpu.sync_copy(mb, agg_sh.at[idx], add=True)

    def body(j, carry):
        o0 = pl.multiple_of(base + (2 * j) * K, 8)
        o1 = pl.multiple_of(base + (2 * j + 1) * K, 8)
        start(o0, K, idx0, mb0, sem0)
        start(o1, K, idx1, mb1, sem1)
        finish(o0, K, idx0, mb0, sem0)
        finish(o1, K, idx1, mb1, sem1)
        return carry

    lax.fori_loop(0, NPAIR, body, 0)
    ot = pl.multiple_of(TAIL0 + wid * KT, 8)
    mbt = mb0.at[pl.ds(0, KT)]
    start(ot, KT, idxt, mbt, sem0)
    finish(ot, KT, idxt, mbt, sem0)
    plsc.subcore_barrier()
    pltpu.sync_copy(agg_sh.at[pl.ds(s * ROWS_PT, ROWS_PT)],
                    out_ref.at[c, pl.ds(s * ROWS_PT, ROWS_PT)])


@functools.lru_cache(maxsize=None)
def _sc_calls():
    mesh = plsc.VectorSubcoreMesh(core_axis_name="c", subcore_axis_name="s",
                                  num_cores=NC, num_subcores=NS)
    gather = pl.kernel(
        _gather_body,
        out_type=(jax.ShapeDtypeStruct((EH, H), jnp.float32),
                  jax.ShapeDtypeStruct((EH, H), jnp.float32)),
        mesh=mesh,
        scratch_types=[
            pltpu.VMEM((K,), jnp.int32),
            pltpu.VMEM((K,), jnp.int32),
            pltpu.VMEM((K,), jnp.int32),
            pltpu.VMEM((K,), jnp.int32),
            pltpu.VMEM((KT,), jnp.int32),
            pltpu.VMEM((KT,), jnp.int32),
            pltpu.VMEM((K, H), jnp.float32),
            pltpu.VMEM((K, H), jnp.float32),
            pltpu.VMEM((K, H), jnp.float32),
            pltpu.VMEM((K, H), jnp.float32),
            pltpu.SemaphoreType.DMA,
            pltpu.SemaphoreType.DMA,
        ],
    )
    scatter = pl.kernel(
        _scatter_body,
        out_type=jax.ShapeDtypeStruct((NC, NP, H), jnp.float32),
        mesh=mesh,
        scratch_types=[
            pltpu.VMEM((K,), jnp.int32),
            pltpu.VMEM((K,), jnp.int32),
            pltpu.VMEM((KT,), jnp.int32),
            pltpu.VMEM((K, H), jnp.float32),
            pltpu.VMEM((K, H), jnp.float32),
            pltpu.SemaphoreType.DMA,
            pltpu.SemaphoreType.DMA,
            pltpu.VMEM_SHARED((NP, H), jnp.float32),
        ],
    )
    return gather, scatter


_lin0_call = pl.pallas_call(
    _lin0_body,
    out_shape=jax.ShapeDtypeStruct((N, H), jnp.float32),
)

_BE = 1600

_msg_call = pl.pallas_call(
    _msg_body,
    grid=(EH // _BE,),
    in_specs=[
        pl.BlockSpec((_BE, H), lambda i: (i, 0)),
        pl.BlockSpec((_BE, H), lambda i: (i, 0)),
        pl.BlockSpec((_BE, DEP), lambda i: (i, 0)),
        pl.BlockSpec((H, 2 * H), lambda i: (0, 0)),
        pl.BlockSpec((H, 2 * H), lambda i: (0, 0)),
        pl.BlockSpec((DEP, 2 * H), lambda i: (0, 0)),
        pl.BlockSpec((1, 2 * H), lambda i: (0, 0)),
    ],
    out_specs=pl.BlockSpec((_BE, H), lambda i: (i, 0)),
    out_shape=jax.ShapeDtypeStruct((EH, H), jnp.float32),
)

_update_call = pl.pallas_call(
    _update_body,
    out_shape=jax.ShapeDtypeStruct((N, H), jnp.float32),
)

_final_call = pl.pallas_call(
    _final_body,
    out_shape=jax.ShapeDtypeStruct((G, 1), jnp.float32),
)


def kernel(x, edge_index, edge_attr, batch, Wp, bp, Wf, bf, Ws, bs,
           g_in, b_in, g_out, b_out, W1, b1, W2, b2):
    srcs = [edge_index[0, :EH].astype(jnp.int32),
            edge_index[0, EH:].astype(jnp.int32)]
    dsts = [edge_index[1, :EH].astype(jnp.int32),
            edge_index[1, EH:].astype(jnp.int32)]
    eas = [edge_attr[:EH], edge_attr[EH:]]
    eas = [jnp.pad(e, ((0, 0), (0, DEP - DE))) for e in eas]
    zeros = jnp.zeros((NP, H), jnp.float32)
    batch_row = batch.astype(jnp.int32).reshape(1, N)

    wd = [jnp.concatenate([Wf[l][:H], Ws[l][:H]], axis=1).astype(jnp.bfloat16)
          for l in range(L)]
    wsr = [jnp.concatenate([Wf[l][H:2 * H], Ws[l][H:2 * H]],
                           axis=1).astype(jnp.bfloat16) for l in range(L)]
    we = [jnp.pad(jnp.concatenate([Wf[l][2 * H:], Ws[l][2 * H:]], axis=1),
                  ((0, DEP - DE), (0, 0))) for l in range(L)]
    bc = [jnp.concatenate([bf[l], bs[l]]).reshape(1, 2 * H) for l in range(L)]

    h = _lin0_call(x, Wp, bp.reshape(1, H))
    _gather_call, _scatter_call = _sc_calls()

    o = None
    for l in range(L):
        parts = [None, None]
        td0, ts0 = _gather_call(dsts[0], srcs[0], h)
        td1, ts1 = _gather_call(dsts[1], srcs[1], h)
        m0 = _msg_call(td0, ts0, eas[0], wd[l], wsr[l], we[l], bc[l])
        parts[0] = _scatter_call(dsts[0], m0, zeros)
        m1 = _msg_call(td1, ts1, eas[1], wd[l], wsr[l], we[l], bc[l])
        parts[1] = _scatter_call(dsts[1], m1, zeros)
        gi = g_in[l].reshape(1, H)
        bi = b_in[l].reshape(1, H)
        go = g_out[l].reshape(1, H)
        bo = b_out[l].reshape(1, H)
        if l < L - 1:
            h = _update_call(h, parts[0], parts[1], gi, bi, go, bo)
        else:
            o = _final_call(h, parts[0], parts[1], gi, bi, go, bo, batch_row,
                            W1, b1.reshape(1, H // 2), W2, b2.reshape(1, 1))
    return o


# speedup vs baseline: 1.1118x; 1.0766x over previous
import functools

import jax
import jax.numpy as jnp
from jax import lax
from jax.experimental import pallas as pl
from jax.experimental.pallas import tpu as pltpu
from jax.experimental.pallas import tpu_sc as plsc

N = 10000
E = 320000
H = 128
HP = H // 2
L = 3
DE = 13
DEP = 16
G = 64

NC = 2
NS = 16
NW = NC * NS
EPT = E // NW
EH = E // 2
K = 128
NCH = 39
NRING = 13
EPT_MAIN = K * NCH
KT = 8
TAIL0 = NW * EPT_MAIN
K2 = 64
NCH2 = 78
NRING2 = 26
NP = 10240
ROWS_PT = NP // NS

_HI = jax.lax.Precision.HIGHEST


def _lin0_body(x_ref, wp_ref, bp_ref, h_ref):
    h_ref[...] = jnp.dot(x_ref[...], wp_ref[...]) + bp_ref[...]


def _msg_body(hd_ref, hs_ref, ea_ref, wd_ref, ws_ref, we_ref, bc_ref, m_ref):
    hd16 = hd_ref[...].astype(jnp.bfloat16)
    hs16 = hs_ref[...].astype(jnp.bfloat16)
    u = (jnp.dot(hd16, wd_ref[...], preferred_element_type=jnp.float32)
         + jnp.dot(hs16, ws_ref[...], preferred_element_type=jnp.float32)
         + jnp.dot(ea_ref[...], we_ref[...]) + bc_ref[...])
    uf = u[:, :H]
    us = u[:, H:]
    sg = 1.0 / (1.0 + jnp.exp(-uf))
    sp = jnp.maximum(us, 0.0) + jnp.log1p(jnp.exp(-jnp.abs(us)))
    m_ref[...] = sg * sp


def _bn(v, g, b):
    mu = jnp.mean(v, axis=0, keepdims=True)
    var = jnp.mean((v - mu) * (v - mu), axis=0, keepdims=True)
    return g * (v - mu) / jnp.sqrt(var + 1e-5) + b


def _update_body(h_ref, pa_ref, pb_ref, gi_ref, bi_ref, go_ref, bo_ref, h2_ref):
    agg = _bn((pa_ref[0, :N] + pa_ref[1, :N]) + (pb_ref[0, :N] + pb_ref[1, :N]),
              gi_ref[...], bi_ref[...])
    h2_ref[...] = jnp.maximum(_bn(h_ref[...] + agg, go_ref[...], bo_ref[...]), 0.0)


def _final_body(h_ref, pa_ref, pb_ref, gi_ref, bi_ref, go_ref, bo_ref, batch_ref,
                w1_ref, b1_ref, w2_ref, b2_ref, o_ref):
    agg = _bn((pa_ref[0, :N] + pa_ref[1, :N]) + (pb_ref[0, :N] + pb_ref[1, :N]),
              gi_ref[...], bi_ref[...])
    h = jnp.maximum(_bn(h_ref[...] + agg, go_ref[...], bo_ref[...]), 0.0)
    seg = lax.broadcasted_iota(jnp.int32, (G, N), 0)
    oht = (seg == batch_ref[...]).astype(jnp.float32)
    sums = jnp.dot(oht, h, precision=_HI)
    cnt = jnp.sum(oht, axis=1, keepdims=True)
    pooled = sums / jnp.maximum(cnt, 1.0)
    o1 = jnp.maximum(jnp.dot(pooled, w1_ref[...]) + b1_ref[...], 0.0)
    o_ref[...] = jnp.dot(o1, w2_ref[...]) + b2_ref[...]


def _gather_body(dst_ref, src_ref, hp_ref, td_ref, ts_ref,
                 idxd, idxs, idxdt, idxst, bd, bs,
                 semi, semg, sems):
    c = lax.axis_index("c")
    s = lax.axis_index("s")
    wid = s * NC + c
    base = wid * EPT_MAIN

    def body(i, carry):
        offs = [pl.multiple_of(base + (3 * i + r) * K, 8) for r in range(3)]
        prev = [base + (3 * i - 3 + r) * K for r in range(3)]
        for r in range(3):
            pltpu.async_copy(dst_ref.at[pl.ds(offs[r], K)], idxd[r], semi[r])
            pltpu.async_copy(src_ref.at[pl.ds(offs[r], K)], idxs[r], semi[r])
        for r in range(3):
            @pl.when(i > 0)
            def _(r=r):
                po = pl.multiple_of(prev[r], 8)
                pltpu.make_async_copy(bd[r], td_ref.at[pl.ds(po, K)], sems[r]).wait()
                pltpu.make_async_copy(bs[r], ts_ref.at[pl.ds(po, K)], sems[r]).wait()
            pltpu.make_async_copy(dst_ref.at[pl.ds(offs[r], K)], idxd[r], semi[r]).wait()
            pltpu.make_async_copy(src_ref.at[pl.ds(offs[r], K)], idxs[r], semi[r]).wait()
            pltpu.async_copy(hp_ref.at[idxd[r]], bd[r], semg[r])
            pltpu.async_copy(hp_ref.at[idxs[r]], bs[r], semg[r])
        for r in range(3):
            pltpu.make_async_copy(hp_ref.at[idxd[r]], bd[r], semg[r]).wait()
            pltpu.make_async_copy(hp_ref.at[idxs[r]], bs[r], semg[r]).wait()
            pltpu.async_copy(bd[r], td_ref.at[pl.ds(offs[r], K)], sems[r])
            pltpu.async_copy(bs[r], ts_ref.at[pl.ds(offs[r], K)], sems[r])
        return carry

    lax.fori_loop(0, NRING, body, 0)
    for r in range(3):
        lo = pl.multiple_of(base + (NCH - 3 + r) * K, 8)
        pltpu.make_async_copy(bd[r], td_ref.at[pl.ds(lo, K)], sems[r]).wait()
        pltpu.make_async_copy(bs[r], ts_ref.at[pl.ds(lo, K)], sems[r]).wait()
    ot = pl.multiple_of(TAIL0 + wid * KT, 8)
    pltpu.sync_copy(dst_ref.at[pl.ds(ot, KT)], idxdt)
    pltpu.sync_copy(src_ref.at[pl.ds(ot, KT)], idxst)
    bdt = bd[0].at[pl.ds(0, KT)]
    bst = bs[0].at[pl.ds(0, KT)]
    pltpu.async_copy(hp_ref.at[idxdt], bdt, semg[0])
    pltpu.async_copy(hp_ref.at[idxst], bst, semg[0])
    pltpu.make_async_copy(hp_ref.at[idxdt], bdt, semg[0]).wait()
    pltpu.make_async_copy(hp_ref.at[idxst], bst, semg[0]).wait()
    pltpu.sync_copy(bdt, td_ref.at[pl.ds(ot, KT)])
    pltpu.sync_copy(bst, ts_ref.at[pl.ds(ot, KT)])


def _scatter_body(dst_ref, m_ref, z_ref, out_ref, idx, idxt, mb, semi, semm, agg_sh):
    c = lax.axis_index("c")
    s = lax.axis_index("s")
    wid = s * NC + c
    pltpu.sync_copy(z_ref.at[pl.ds(s * ROWS_PT, ROWS_PT)],
                    agg_sh.at[pl.ds(s * ROWS_PT, ROWS_PT)])
    plsc.subcore_barrier()
    base = wid * EPT_MAIN

    def body(i, carry):
        offs = [pl.multiple_of(base + (3 * i + r) * K2, 8) for r in range(3)]
        for r in range(3):
            pltpu.async_copy(dst_ref.at[pl.ds(offs[r], K2)], idx[r], semi[r])
            pltpu.async_copy(m_ref.at[pl.ds(offs[r], K2)], mb[r], semm[r])
        for r in range(3):
            pltpu.make_async_copy(dst_ref.at[pl.ds(offs[r], K2)], idx[r], semi[r]).wait()
            pltpu.make_async_copy(m_ref.at[pl.ds(offs[r], K2)], mb[r], semm[r]).wait()
            pltpu.sync_copy(mb[r], agg_sh.at[idx[r]], add=True)
        return carry

    lax.fori_loop(0, NRING2, body, 0)
    ot = pl.multiple_of(TAIL0 + wid * KT, 8)
    pltpu.sync_copy(dst_ref.at[pl.ds(ot, KT)], idxt)
    mbt = mb[0].at[pl.ds(0, KT)]
    pltpu.sync_copy(m_ref.at[pl.ds(ot, KT)], mbt)
    pltpu.sync_copy(mbt, agg_sh.at[idxt], add=True)
    plsc.subcore_barrier()
    pltpu.sync_copy(agg_sh.at[pl.ds(s * ROWS_PT, ROWS_PT)],
                    out_ref.at[c, pl.ds(s * ROWS_PT, ROWS_PT)])


@functools.lru_cache(maxsize=None)
def _sc_calls():
    mesh = plsc.VectorSubcoreMesh(core_axis_name="c", subcore_axis_name="s",
                                  num_cores=NC, num_subcores=NS)
    gather = pl.kernel(
        _gather_body,
        out_type=(jax.ShapeDtypeStruct((EH, H), jnp.float32),
                  jax.ShapeDtypeStruct((EH, H), jnp.float32)),
        mesh=mesh,
        scratch_types=[
            [pltpu.VMEM((K,), jnp.int32)] * 3,
            [pltpu.VMEM((K,), jnp.int32)] * 3,
            pltpu.VMEM((KT,), jnp.int32),
            pltpu.VMEM((KT,), jnp.int32),
            [pltpu.VMEM((K, H), jnp.float32)] * 3,
            [pltpu.VMEM((K, H), jnp.float32)] * 3,
            [pltpu.SemaphoreType.DMA] * 3,
            [pltpu.SemaphoreType.DMA] * 3,
            [pltpu.SemaphoreType.DMA] * 3,
        ],
    )
    scatter = pl.kernel(
        _scatter_body,
        out_type=jax.ShapeDtypeStruct((NC, NP, H), jnp.float32),
        mesh=mesh,
        scratch_types=[
            [pltpu.VMEM((K2,), jnp.int32)] * 3,
            pltpu.VMEM((KT,), jnp.int32),
            [pltpu.VMEM((K2, H), jnp.float32)] * 3,
            [pltpu.SemaphoreType.DMA] * 3,
            [pltpu.SemaphoreType.DMA] * 3,
            pltpu.VMEM_SHARED((NP, H), jnp.float32),
        ],
    )
    return gather, scatter


_lin0_call = pl.pallas_call(
    _lin0_body,
    out_shape=jax.ShapeDtypeStruct((N, H), jnp.float32),
)

_BE = 1600

_msg_call = pl.pallas_call(
    _msg_body,
    grid=(EH // _BE,),
    in_specs=[
        pl.BlockSpec((_BE, H), lambda i: (i, 0)),
        pl.BlockSpec((_BE, H), lambda i: (i, 0)),
        pl.BlockSpec((_BE, DEP), lambda i: (i, 0)),
        pl.BlockSpec((H, 2 * H), lambda i: (0, 0)),
        pl.BlockSpec((H, 2 * H), lambda i: (0, 0)),
        pl.BlockSpec((DEP, 2 * H), lambda i: (0, 0)),
        pl.BlockSpec((1, 2 * H), lambda i: (0, 0)),
    ],
    out_specs=pl.BlockSpec((_BE, H), lambda i: (i, 0)),
    out_shape=jax.ShapeDtypeStruct((EH, H), jnp.float32),
)

_update_call = pl.pallas_call(
    _update_body,
    out_shape=jax.ShapeDtypeStruct((N, H), jnp.float32),
)

_final_call = pl.pallas_call(
    _final_body,
    out_shape=jax.ShapeDtypeStruct((G, 1), jnp.float32),
)


def kernel(x, edge_index, edge_attr, batch, Wp, bp, Wf, bf, Ws, bs,
           g_in, b_in, g_out, b_out, W1, b1, W2, b2):
    srcs = [edge_index[0, :EH].astype(jnp.int32),
            edge_index[0, EH:].astype(jnp.int32)]
    dsts = [edge_index[1, :EH].astype(jnp.int32),
            edge_index[1, EH:].astype(jnp.int32)]
    eas = [edge_attr[:EH], edge_attr[EH:]]
    eas = [jnp.pad(e, ((0, 0), (0, DEP - DE))) for e in eas]
    zeros = jnp.zeros((NP, H), jnp.float32)
    batch_row = batch.astype(jnp.int32).reshape(1, N)

    wd = [jnp.concatenate([Wf[l][:H], Ws[l][:H]], axis=1).astype(jnp.bfloat16)
          for l in range(L)]
    wsr = [jnp.concatenate([Wf[l][H:2 * H], Ws[l][H:2 * H]],
                           axis=1).astype(jnp.bfloat16) for l in range(L)]
    we = [jnp.pad(jnp.concatenate([Wf[l][2 * H:], Ws[l][2 * H:]], axis=1),
                  ((0, DEP - DE), (0, 0))) for l in range(L)]
    bc = [jnp.concatenate([bf[l], bs[l]]).reshape(1, 2 * H) for l in range(L)]

    h = _lin0_call(x, Wp, bp.reshape(1, H))
    _gather_call, _scatter_call = _sc_calls()

    o = None
    for l in range(L):
        parts = [None, None]
        td0, ts0 = _gather_call(dsts[0], srcs[0], h)
        td1, ts1 = _gather_call(dsts[1], srcs[1], h)
        m0 = _msg_call(td0, ts0, eas[0], wd[l], wsr[l], we[l], bc[l])
        parts[0] = _scatter_call(dsts[0], m0, zeros)
        m1 = _msg_call(td1, ts1, eas[1], wd[l], wsr[l], we[l], bc[l])
        parts[1] = _scatter_call(dsts[1], m1, zeros)
        gi = g_in[l].reshape(1, H)
        bi = b_in[l].reshape(1, H)
        go = g_out[l].reshape(1, H)
        bo = b_out[l].reshape(1, H)
        if l < L - 1:
            h = _update_call(h, parts[0], parts[1], gi, bi, go, bo)
        else:
            o = _final_call(h, parts[0], parts[1], gi, bi, go, bo, batch_row,
                            W1, b1.reshape(1, H // 2), W2, b2.reshape(1, 1))
    return o
